# baseline (speedup 1.0000x reference)
import functools

import jax
import jax.numpy as jnp
from jax import lax
from jax.experimental import pallas as pl
from jax.experimental.pallas import tpu as pltpu
from jax.experimental.pallas import tpu_sc as plsc

B, N, NK, NN = 8, 10000, 64, 3
M = B * N
NC, NS = 2, 16
NW = NC * NS
CP = 80
NCHK = M // CP
RMAX = 32
M_PAD = NW * RMAX * CP
LANES = NK // 16
NKP = 128


def _project(normals, dirs):
    def body(n_ref, d_ref, o_ref):
        n = n_ref[0]
        d = d_ref[...]
        dn = d / jnp.maximum(
            jnp.sqrt(jnp.sum(d * d, axis=0, keepdims=True)), 1e-12)
        inv = 1.0 / jnp.maximum(
            jnp.sqrt(jnp.sum(n * n, axis=0, keepdims=True)), 1e-12)
        nh = n * inv
        o_ref[:, :NK] = lax.dot_general(
            nh, dn, (((0,), (0,)), ((), ())),
            preferred_element_type=jnp.float32)
        o_ref[:, NK:] = jnp.zeros((N, NKP - NK), jnp.float32)

    out = pl.pallas_call(
        body,
        grid=(B,),
        in_specs=[pl.BlockSpec((1, 3, N), lambda b: (b, 0, 0)),
                  pl.BlockSpec((3, NK), lambda b: (0, 0))],
        out_specs=pl.BlockSpec((N, NKP), lambda b: (b, 0)),
        out_shape=jax.ShapeDtypeStruct((M_PAD, NKP), jnp.float32),
    )(normals, dirs)
    return out


def _gather_max(proj, idx):
    mesh = plsc.VectorSubcoreMesh(core_axis_name="c", subcore_axis_name="s",
                                  num_cores=NC, num_subcores=NS)

    @functools.partial(
        pl.kernel,
        mesh=mesh,
        compiler_params=pltpu.CompilerParams(use_tc_tiling_on_sc=False),
        out_type=(jax.ShapeDtypeStruct((M_PAD, NKP), jnp.float32),
                  jax.ShapeDtypeStruct((NW, 2 * NK), jnp.float32)),
        scratch_types=[
            pltpu.VMEM((RMAX, NN, CP), jnp.int32),
            pltpu.VMEM((2, NN, CP, NKP), jnp.float32),
            pltpu.VMEM((2, CP, NKP), jnp.float32),
            pltpu.VMEM((2, CP, NKP), jnp.float32),
            pltpu.VMEM((2 * NK,), jnp.float32),
            pltpu.SemaphoreType.DMA,
            pltpu.SemaphoreType.DMA,
            pltpu.SemaphoreType.DMA,
            pltpu.SemaphoreType.DMA,
        ],
    )
    def body(proj_hbm, idx_hbm, feat_hbm, part_hbm,
             idxv, gv, sv, fv, pv, gsem0, gsem1, wsem0, wsem1):
        wid = lax.axis_index("s") * NC + lax.axis_index("c")
        zero = jnp.zeros((16,), jnp.float32)
        zacc = (zero,) * (2 * LANES)
        gsem = (gsem0, gsem1)
        wsem = (wsem0, wsem1)

        def gather_cps(t, p):
            cps = [pltpu.make_async_copy(
                proj_hbm.at[idxv.at[t, k]], gv.at[p, k], gsem[p])
                for k in range(NN)]
            base = (wid + t * NW) * CP
            cps.append(pltpu.make_async_copy(
                proj_hbm.at[pl.ds(base, CP)], sv.at[p], gsem[p]))
            return cps

        def fire(t, p):
            for cp in gather_cps(t, p):
                cp.start()

        def wait_fire(t, p):
            for cp in gather_cps(t, p):
                cp.wait()

        def write_cp(t, p):
            base = (wid + t * NW) * CP
            return pltpu.make_async_copy(
                fv.at[p], feat_hbm.at[pl.ds(base, CP)], wsem[p])

        def round_acc(p):
            def face(j, a):
                new = list(a)
                for c in range(LANES):
                    sl = pl.ds(c * 16, 16)
                    f = jnp.maximum(
                        jnp.maximum(gv[p, 0, j, sl], gv[p, 1, j, sl]),
                        jnp.maximum(gv[p, 2, j, sl], sv[p, j, sl]))
                    fv[p, j, sl] = f
                    new[2 * c] = a[2 * c] + f
                    new[2 * c + 1] = a[2 * c + 1] + f * f
                return tuple(new)

            return lax.fori_loop(0, CP, face, zacc)

        def addt(a, b):
            return tuple(x + y for x, y in zip(a, b))

        pltpu.sync_copy(idx_hbm.at[wid], idxv)
        fire(0, 0)
        fire(1, 1)

        wait_fire(0, 0)
        acc = round_acc(0)
        fire(2, 0)
        write_cp(0, 0).start()
        wait_fire(1, 1)
        acc = addt(acc, round_acc(1))
        fire(3, 1)
        write_cp(1, 1).start()

        def pairbody(kk, acc):
            t = 2 * kk
            wait_fire(t, 0)
            write_cp(0, 0).wait()
            acc = addt(acc, round_acc(0))
            fire(t + 2, 0)
            write_cp(t, 0).start()
            wait_fire(t + 1, 1)
            write_cp(0, 1).wait()
            acc = addt(acc, round_acc(1))
            fire(t + 3, 1)
            write_cp(t + 1, 1).start()
            return acc

        acc = lax.fori_loop(1, RMAX // 2 - 1, pairbody, acc)

        wait_fire(RMAX - 2, 0)
        write_cp(0, 0).wait()
        acc = addt(acc, round_acc(0))
        write_cp(RMAX - 2, 0).start()

        live = (wid + (RMAX - 1) * NW) < NCHK
        wait_fire(RMAX - 1, 1)
        write_cp(0, 1).wait()
        last = round_acc(1)
        write_cp(RMAX - 1, 1).start()
        acc = tuple(a + jnp.where(live, x, zero) for a, x in zip(acc, last))

        write_cp(0, 0).wait()
        write_cp(0, 1).wait()

        for c in range(LANES):
            pv[pl.ds(c * 16, 16)] = acc[2 * c]
            pv[pl.ds(NK + c * 16, 16)] = acc[2 * c + 1]
        pltpu.sync_copy(pv, part_hbm.at[wid])

    return body(proj, idx)


def _bn_relu_transpose(feat, part, gamma, beta):
    def body(f_ref, p_ref, g_ref, b_ref, o_ref):
        tot = jnp.sum(p_ref[...], axis=0, keepdims=True)
        s = tot[:, :NK]
        sq = tot[:, NK:]
        mean = s / M
        var = sq / M - mean * mean
        scale = g_ref[...] / jnp.sqrt(var + 1e-5)
        bias = b_ref[...] - mean * scale
        y = jnp.maximum(f_ref[:, :NK] * scale + bias, 0.0)
        o_ref[0] = jnp.transpose(y)

    return pl.pallas_call(
        body,
        grid=(B,),
        in_specs=[pl.BlockSpec((N, NKP), lambda i: (i, 0)),
                  pl.BlockSpec((NW, 2 * NK), lambda i: (0, 0)),
                  pl.BlockSpec((1, NK), lambda i: (0, 0)),
                  pl.BlockSpec((1, NK), lambda i: (0, 0))],
        out_specs=pl.BlockSpec((1, NK, N), lambda i: (i, 0, 0)),
        out_shape=jax.ShapeDtypeStruct((B, NK, N), jnp.float32),
    )(feat, part, gamma, beta)


def kernel(normals, ring_n, directions, gamma, beta):
    offs = (jnp.arange(B, dtype=jnp.int32) * N)[:, None, None]
    gidx = jnp.transpose(ring_n + offs, (2, 0, 1)).reshape(NN, NCHK, CP)
    chunks = jnp.transpose(gidx, (1, 0, 2))
    chunks = jnp.pad(chunks, ((0, NW * RMAX - NCHK), (0, 0), (0, 0)))
    idx = jnp.transpose(chunks.reshape(RMAX, NW, NN, CP), (1, 0, 2, 3))

    proj = _project(normals, directions)
    feat, part = _gather_max(proj, idx)
    return _bn_relu_transpose(feat, part,
                              gamma.reshape(1, NK), beta.reshape(1, NK))


# speedup vs baseline: 2.0629x; 2.0629x over previous
import functools

import jax
import jax.numpy as jnp
from jax import lax
from jax.experimental import pallas as pl
from jax.experimental.pallas import tpu as pltpu
from jax.experimental.pallas import tpu_sc as plsc

B, N, NK, NN = 8, 10000, 64, 3
M = B * N
NC, NS = 2, 16
NW = NC * NS
CP = 160
NCHK = M // CP
LANES = NK // 16
NKP = 128


def _project(normals, dirs):
    def body(n_ref, d_ref, o_ref):
        n = n_ref[0]
        d = d_ref[...]
        dn = d / jnp.maximum(
            jnp.sqrt(jnp.sum(d * d, axis=0, keepdims=True)), 1e-12)
        inv = 1.0 / jnp.maximum(
            jnp.sqrt(jnp.sum(n * n, axis=0, keepdims=True)), 1e-12)
        nh = n * inv
        o_ref[:, :NK] = lax.dot_general(
            nh, dn, (((0,), (0,)), ((), ())),
            preferred_element_type=jnp.float32)
        o_ref[:, NK:] = jnp.zeros((N, NKP - NK), jnp.float32)

    out = pl.pallas_call(
        body,
        grid=(B,),
        in_specs=[pl.BlockSpec((1, 3, N), lambda b: (b, 0, 0)),
                  pl.BlockSpec((3, NK), lambda b: (0, 0))],
        out_specs=pl.BlockSpec((N, NKP), lambda b: (b, 0)),
        out_shape=jax.ShapeDtypeStruct((M, NKP), jnp.float32),
    )(normals, dirs)
    return out


def _gather_max(proj, idx):
    mesh = plsc.VectorSubcoreMesh(core_axis_name="c", subcore_axis_name="s",
                                  num_cores=NC, num_subcores=NS)

    @functools.partial(
        pl.kernel,
        mesh=mesh,
        compiler_params=pltpu.CompilerParams(use_tc_tiling_on_sc=False),
        out_type=(jax.ShapeDtypeStruct((M, NKP), jnp.float32),
                  jax.ShapeDtypeStruct((NW, 2 * NK), jnp.float32)),
        scratch_types=[
            pltpu.VMEM((NN, CP), jnp.int32),
            pltpu.VMEM((CP, NKP), jnp.float32),
            pltpu.VMEM((CP, NKP), jnp.float32),
            pltpu.VMEM((CP, NKP), jnp.float32),
            pltpu.VMEM((CP, NKP), jnp.float32),
            pltpu.VMEM((CP, NKP), jnp.float32),
            pltpu.VMEM((2 * NK,), jnp.float32),
            pltpu.SemaphoreType.DMA,
            pltpu.SemaphoreType.DMA,
        ],
    )
    def body(proj_hbm, idx_hbm, feat_hbm, part_hbm,
             idx_v, g0, g1, g2, selfv, featv, pv, sem, wsem):
        wid = lax.axis_index("s") * NC + lax.axis_index("c")
        zero = jnp.zeros((16,), jnp.float32)
        n_rounds = (NCHK - wid + NW - 1) // NW
        pltpu.async_copy(featv, feat_hbm.at[pl.ds(wid * CP, CP)], wsem)

        def round_body(t, acc):
            chunk = wid + t * NW
            base = chunk * CP
            pltpu.sync_copy(idx_hbm.at[chunk], idx_v)
            cp0 = pltpu.async_copy(proj_hbm.at[idx_v.at[0]], g0, sem)
            cp1 = pltpu.async_copy(proj_hbm.at[idx_v.at[1]], g1, sem)
            cp2 = pltpu.async_copy(proj_hbm.at[idx_v.at[2]], g2, sem)
            pltpu.sync_copy(proj_hbm.at[pl.ds(base, CP)], selfv)
            pltpu.make_async_copy(
                featv, feat_hbm.at[pl.ds(wid * CP, CP)], wsem).wait()
            cp0.wait()
            cp1.wait()
            cp2.wait()

            def face(j, a):
                new = list(a)
                for c in range(LANES):
                    sl = pl.ds(c * 16, 16)
                    f = jnp.maximum(
                        jnp.maximum(g0[j, sl], g1[j, sl]),
                        jnp.maximum(g2[j, sl], selfv[j, sl]))
                    featv[j, sl] = f
                    new[2 * c] = a[2 * c] + f
                    new[2 * c + 1] = a[2 * c + 1] + f * f
                return tuple(new)

            acc = lax.fori_loop(0, CP, face, acc)
            pltpu.async_copy(featv, feat_hbm.at[pl.ds(base, CP)], wsem)
            return acc

        acc = lax.fori_loop(0, n_rounds, round_body, (zero,) * (2 * LANES))
        pltpu.make_async_copy(
            featv, feat_hbm.at[pl.ds(wid * CP, CP)], wsem).wait()
        for c in range(LANES):
            pv[pl.ds(c * 16, 16)] = acc[2 * c]
            pv[pl.ds(NK + c * 16, 16)] = acc[2 * c + 1]
        pltpu.sync_copy(pv, part_hbm.at[wid])

    return body(proj, idx)


def _bn_relu_transpose(feat, part, gamma, beta):
    def body(f_ref, p_ref, g_ref, b_ref, o_ref):
        tot = jnp.sum(p_ref[...], axis=0, keepdims=True)
        s = tot[:, :NK]
        sq = tot[:, NK:]
        mean = s / M
        var = sq / M - mean * mean
        scale = g_ref[...] / jnp.sqrt(var + 1e-5)
        bias = b_ref[...] - mean * scale
        y = jnp.maximum(f_ref[:, :NK] * scale + bias, 0.0)
        o_ref[0] = jnp.transpose(y)

    return pl.pallas_call(
        body,
        grid=(B,),
        in_specs=[pl.BlockSpec((N, NKP), lambda i: (i, 0)),
                  pl.BlockSpec((NW, 2 * NK), lambda i: (0, 0)),
                  pl.BlockSpec((1, NK), lambda i: (0, 0)),
                  pl.BlockSpec((1, NK), lambda i: (0, 0))],
        out_specs=pl.BlockSpec((1, NK, N), lambda i: (i, 0, 0)),
        out_shape=jax.ShapeDtypeStruct((B, NK, N), jnp.float32),
    )(feat, part, gamma, beta)


def kernel(normals, ring_n, directions, gamma, beta):
    offs = (jnp.arange(B, dtype=jnp.int32) * N)[:, None, None]
    gidx = jnp.transpose(ring_n + offs, (2, 0, 1)).reshape(NN, NCHK, CP)
    idx = jnp.transpose(gidx, (1, 0, 2))

    proj = _project(normals, directions)
    feat, part = _gather_max(proj, idx)
    return _bn_relu_transpose(feat, part,
                              gamma.reshape(1, NK), beta.reshape(1, NK))


# speedup vs baseline: 2.1714x; 1.0526x over previous
import functools

import jax
import jax.numpy as jnp
from jax import lax
from jax.experimental import pallas as pl
from jax.experimental.pallas import tpu as pltpu
from jax.experimental.pallas import tpu_sc as plsc

B, N, NK, NN = 8, 10000, 64, 3
M = B * N
NC, NS = 2, 16
NW = NC * NS
CP = 160
NCHK = M // CP
RT = (NCHK + NW - 1) // NW
LANES = NK // 16
NKP = 128


def _project(normals, dirs):
    def body(n_ref, d_ref, o_ref):
        n = n_ref[0]
        d = d_ref[...]
        dn = d / jnp.maximum(
            jnp.sqrt(jnp.sum(d * d, axis=0, keepdims=True)), 1e-12)
        inv = 1.0 / jnp.maximum(
            jnp.sqrt(jnp.sum(n * n, axis=0, keepdims=True)), 1e-12)
        nh = n * inv
        o_ref[:, :NK] = lax.dot_general(
            nh, dn, (((0,), (0,)), ((), ())),
            preferred_element_type=jnp.float32)
        o_ref[:, NK:] = jnp.zeros((N, NKP - NK), jnp.float32)

    out = pl.pallas_call(
        body,
        grid=(B,),
        in_specs=[pl.BlockSpec((1, 3, N), lambda b: (b, 0, 0)),
                  pl.BlockSpec((3, NK), lambda b: (0, 0))],
        out_specs=pl.BlockSpec((N, NKP), lambda b: (b, 0)),
        out_shape=jax.ShapeDtypeStruct((M, NKP), jnp.float32),
    )(normals, dirs)
    return out


def _gather_max(proj, idx):
    mesh = plsc.VectorSubcoreMesh(core_axis_name="c", subcore_axis_name="s",
                                  num_cores=NC, num_subcores=NS)

    @functools.partial(
        pl.kernel,
        mesh=mesh,
        compiler_params=pltpu.CompilerParams(use_tc_tiling_on_sc=False),
        out_type=(jax.ShapeDtypeStruct((M, NKP), jnp.float32),
                  jax.ShapeDtypeStruct((NW, 2 * NK), jnp.float32)),
        scratch_types=[
            pltpu.VMEM((RT, NN, CP), jnp.int32),
            pltpu.VMEM((CP, NKP), jnp.float32),
            pltpu.VMEM((CP, NKP), jnp.float32),
            pltpu.VMEM((CP, NKP), jnp.float32),
            pltpu.VMEM((CP, NKP), jnp.float32),
            pltpu.VMEM((CP, NKP), jnp.float32),
            pltpu.VMEM((2 * NK,), jnp.float32),
            pltpu.SemaphoreType.DMA,
            pltpu.SemaphoreType.DMA,
        ],
    )
    def body(proj_hbm, idx_hbm, feat_hbm, part_hbm,
             idx_v, g0, g1, g2, selfv, featv, pv, sem, wsem):
        wid = lax.axis_index("s") * NC + lax.axis_index("c")
        zero = jnp.zeros((16,), jnp.float32)
        n_rounds = (NCHK - wid + NW - 1) // NW
        pltpu.sync_copy(idx_hbm.at[wid], idx_v)
        pltpu.async_copy(featv, feat_hbm.at[pl.ds(wid * CP, CP)], wsem)

        def round_body(t, acc):
            chunk = wid + t * NW
            base = chunk * CP
            cp0 = pltpu.async_copy(proj_hbm.at[idx_v.at[t, 0]], g0, sem)
            cp1 = pltpu.async_copy(proj_hbm.at[idx_v.at[t, 1]], g1, sem)
            cp2 = pltpu.async_copy(proj_hbm.at[idx_v.at[t, 2]], g2, sem)
            pltpu.sync_copy(proj_hbm.at[pl.ds(base, CP)], selfv)
            pltpu.make_async_copy(
                featv, feat_hbm.at[pl.ds(wid * CP, CP)], wsem).wait()
            cp0.wait()
            cp1.wait()
            cp2.wait()

            def face(j, a):
                new = list(a)
                for c in range(LANES):
                    sl = pl.ds(c * 16, 16)
                    f = jnp.maximum(
                        jnp.maximum(g0[j, sl], g1[j, sl]),
                        jnp.maximum(g2[j, sl], selfv[j, sl]))
                    featv[j, sl] = f
                    new[2 * c] = a[2 * c] + f
                    new[2 * c + 1] = a[2 * c + 1] + f * f
                return tuple(new)

            acc = lax.fori_loop(0, CP, face, acc)
            pltpu.async_copy(featv, feat_hbm.at[pl.ds(base, CP)], wsem)
            return acc

        acc = lax.fori_loop(0, n_rounds, round_body, (zero,) * (2 * LANES))
        pltpu.make_async_copy(
            featv, feat_hbm.at[pl.ds(wid * CP, CP)], wsem).wait()
        for c in range(LANES):
            pv[pl.ds(c * 16, 16)] = acc[2 * c]
            pv[pl.ds(NK + c * 16, 16)] = acc[2 * c + 1]
        pltpu.sync_copy(pv, part_hbm.at[wid])

    return body(proj, idx)


def _bn_relu_transpose(feat, part, gamma, beta):
    def body(f_ref, p_ref, g_ref, b_ref, o_ref):
        tot = jnp.sum(p_ref[...], axis=0, keepdims=True)
        s = tot[:, :NK]
        sq = tot[:, NK:]
        mean = s / M
        var = sq / M - mean * mean
        scale = g_ref[...] / jnp.sqrt(var + 1e-5)
        bias = b_ref[...] - mean * scale
        y = jnp.maximum(f_ref[:, :NK] * scale + bias, 0.0)
        o_ref[0] = jnp.transpose(y)

    return pl.pallas_call(
        body,
        grid=(B,),
        in_specs=[pl.BlockSpec((N, NKP), lambda i: (i, 0)),
                  pl.BlockSpec((NW, 2 * NK), lambda i: (0, 0)),
                  pl.BlockSpec((1, NK), lambda i: (0, 0)),
                  pl.BlockSpec((1, NK), lambda i: (0, 0))],
        out_specs=pl.BlockSpec((1, NK, N), lambda i: (i, 0, 0)),
        out_shape=jax.ShapeDtypeStruct((B, NK, N), jnp.float32),
    )(feat, part, gamma, beta)


def kernel(normals, ring_n, directions, gamma, beta):
    offs = (jnp.arange(B, dtype=jnp.int32) * N)[:, None, None]
    gidx = jnp.transpose(ring_n + offs, (2, 0, 1)).reshape(NN, NCHK, CP)
    chunks = jnp.transpose(gidx, (1, 0, 2))
    chunks = jnp.pad(chunks, ((0, NW * RT - NCHK), (0, 0), (0, 0)))
    idx = jnp.transpose(chunks.reshape(RT, NW, NN, CP), (1, 0, 2, 3))

    proj = _project(normals, directions)
    feat, part = _gather_max(proj, idx)
    return _bn_relu_transpose(feat, part,
                              gamma.reshape(1, NK), beta.reshape(1, NK))
